# baseline (speedup 1.0000x reference)
import jax
import jax.numpy as jnp
from jax.experimental import pallas as pl
from jax.experimental.pallas import tpu as pltpu


def _build_hamilton(weight):
    a0, a1, a2, a3, a4, a5, a6, a7 = jnp.split(weight, 8, axis=1)
    rows = [
        [a0, a1, a2, a3, a4, a5, a6, a7],
        [a1, -a0, a3, -a2, a5, -a4, -a7, a6],
        [a2, -a3, -a0, a1, a6, a7, -a4, -a5],
        [a3, a2, -a1, -a0, a7, -a6, a5, -a4],
        [a4, -a5, -a6, -a7, -a0, a1, a2, a3],
        [a5, a4, -a7, a6, -a1, -a0, -a3, a2],
        [a6, a7, a4, -a5, -a2, a3, -a0, -a1],
        [a7, -a6, a5, a4, -a3, -a2, a1, -a0],
    ]
    return jnp.concatenate(
        [jnp.concatenate(r, axis=0) for r in rows], axis=1)


def _make_fused(n, out_f, bm):
    nblk = n // bm

    def fused(x_ref, h_ref, g_ref, b_ref, adj_ref, out_ref,
              sup_ref, y_ref, stat_ref):
        s = pl.program_id(0)

        @pl.when(s == 0)
        def _init():
            sup_ref[...] = jnp.dot(x_ref[...], h_ref[...],
                                   preferred_element_type=jnp.float32)
            stat_ref[...] = jnp.zeros_like(stat_ref)

        @pl.when(s < nblk)
        def _compute():
            y = jnp.dot(adj_ref[...], sup_ref[...],
                        preferred_element_type=jnp.float32)
            y_ref[pl.ds(s * bm, bm), :] = y
            stat_ref[0:1, :] += jnp.sum(y, axis=0, keepdims=True)
            stat_ref[1:2, :] += jnp.sum(y * y, axis=0, keepdims=True)

        @pl.when(s >= nblk)
        def _normalize():
            mean = stat_ref[0:1, :] / n
            var = stat_ref[1:2, :] / n - mean * mean
            scale = jax.lax.rsqrt(var + 1e-5) * g_ref[...]
            shift = b_ref[...] - mean * scale
            yb = y_ref[pl.ds((s - nblk) * bm, bm), :]
            out_ref[...] = jnp.tanh(yb * scale + shift)

    return fused


def kernel(input, adj, weight, gamma, beta):
    n, in_f = input.shape
    out_f = weight.shape[1]
    hamilton = _build_hamilton(weight)

    bm = 400
    nblk = n // bm
    return pl.pallas_call(
        _make_fused(n, out_f, bm),
        grid=(2 * nblk,),
        in_specs=[
            pl.BlockSpec((n, in_f), lambda s: (0, 0)),
            pl.BlockSpec((in_f, out_f), lambda s: (0, 0)),
            pl.BlockSpec((1, out_f), lambda s: (0, 0)),
            pl.BlockSpec((1, out_f), lambda s: (0, 0)),
            pl.BlockSpec((bm, n), lambda s: (jnp.minimum(s, nblk - 1), 0)),
        ],
        out_specs=pl.BlockSpec(
            (bm, out_f), lambda s: (jnp.maximum(s - nblk, 0), 0)),
        out_shape=jax.ShapeDtypeStruct((n, out_f), jnp.float32),
        scratch_shapes=[
            pltpu.VMEM((n, out_f), jnp.float32),
            pltpu.VMEM((n, out_f), jnp.float32),
            pltpu.VMEM((8, out_f), jnp.float32),
        ],
    )(input, hamilton, gamma.reshape(1, out_f), beta.reshape(1, out_f), adj)


# speedup vs baseline: 1.0492x; 1.0492x over previous
import jax
import jax.numpy as jnp
from jax.experimental import pallas as pl
from jax.experimental.pallas import tpu as pltpu


def _build_hamilton(weight):
    a0, a1, a2, a3, a4, a5, a6, a7 = jnp.split(weight, 8, axis=1)
    rows = [
        [a0, a1, a2, a3, a4, a5, a6, a7],
        [a1, -a0, a3, -a2, a5, -a4, -a7, a6],
        [a2, -a3, -a0, a1, a6, a7, -a4, -a5],
        [a3, a2, -a1, -a0, a7, -a6, a5, -a4],
        [a4, -a5, -a6, -a7, -a0, a1, a2, a3],
        [a5, a4, -a7, a6, -a1, -a0, -a3, a2],
        [a6, a7, a4, -a5, -a2, a3, -a0, -a1],
        [a7, -a6, a5, a4, -a3, -a2, a1, -a0],
    ]
    return jnp.concatenate(
        [jnp.concatenate(r, axis=0) for r in rows], axis=1)


def _make_fused(n, out_f, bm):
    nblk = n // bm

    def fused(x_ref, h_ref, g_ref, b_ref, adj_ref, out_ref,
              sup_ref, stat_ref):
        i = pl.program_id(0)

        @pl.when(i == 0)
        def _init():
            sup_ref[...] = jnp.dot(x_ref[...], h_ref[...],
                                   preferred_element_type=jnp.float32)
            stat_ref[...] = jnp.zeros_like(stat_ref)

        y = jnp.dot(adj_ref[...], sup_ref[...],
                    preferred_element_type=jnp.float32)
        out_ref[pl.ds(i * bm, bm), :] = y
        stat_ref[0:1, :] += jnp.sum(y, axis=0, keepdims=True)
        stat_ref[1:2, :] += jnp.sum(y * y, axis=0, keepdims=True)

        @pl.when(i == nblk - 1)
        def _epilogue():
            mean = stat_ref[0:1, :] / n
            var = stat_ref[1:2, :] / n - mean * mean
            scale = jax.lax.rsqrt(var + 1e-5) * g_ref[...]
            shift = b_ref[...] - mean * scale

            for j in range(nblk):
                yb = out_ref[j * bm:(j + 1) * bm, :]
                out_ref[j * bm:(j + 1) * bm, :] = jnp.tanh(yb * scale + shift)

    return fused


def kernel(input, adj, weight, gamma, beta):
    n, in_f = input.shape
    out_f = weight.shape[1]
    hamilton = _build_hamilton(weight)

    bm = 400
    nblk = n // bm
    return pl.pallas_call(
        _make_fused(n, out_f, bm),
        grid=(nblk,),
        in_specs=[
            pl.BlockSpec((n, in_f), lambda i: (0, 0)),
            pl.BlockSpec((in_f, out_f), lambda i: (0, 0)),
            pl.BlockSpec((1, out_f), lambda i: (0, 0)),
            pl.BlockSpec((1, out_f), lambda i: (0, 0)),
            pl.BlockSpec((bm, n), lambda i: (i, 0)),
        ],
        out_specs=pl.BlockSpec((n, out_f), lambda i: (0, 0)),
        out_shape=jax.ShapeDtypeStruct((n, out_f), jnp.float32),
        scratch_shapes=[
            pltpu.VMEM((n, out_f), jnp.float32),
            pltpu.VMEM((8, out_f), jnp.float32),
        ],
    )(input, hamilton, gamma.reshape(1, out_f), beta.reshape(1, out_f), adj)


# speedup vs baseline: 1.0859x; 1.0350x over previous
import jax
import jax.numpy as jnp
from jax.experimental import pallas as pl
from jax.experimental.pallas import tpu as pltpu


def _build_hamilton(weight):
    a0, a1, a2, a3, a4, a5, a6, a7 = [
        weight[:, k * (weight.shape[1] // 8):(k + 1) * (weight.shape[1] // 8)]
        for k in range(8)]
    rows = [
        [a0, a1, a2, a3, a4, a5, a6, a7],
        [a1, -a0, a3, -a2, a5, -a4, -a7, a6],
        [a2, -a3, -a0, a1, a6, a7, -a4, -a5],
        [a3, a2, -a1, -a0, a7, -a6, a5, -a4],
        [a4, -a5, -a6, -a7, -a0, a1, a2, a3],
        [a5, a4, -a7, a6, -a1, -a0, -a3, a2],
        [a6, a7, a4, -a5, -a2, a3, -a0, -a1],
        [a7, -a6, a5, a4, -a3, -a2, a1, -a0],
    ]
    return jnp.concatenate(
        [jnp.concatenate(r, axis=0) for r in rows], axis=1)


def _make_fused(n, out_f, bm):
    nblk = n // bm

    def fused(x_ref, w_ref, g_ref, b_ref, adj_ref, out_ref,
              sup_ref, stat_ref):
        i = pl.program_id(0)

        @pl.when(i == 0)
        def _init():
            hamilton = _build_hamilton(w_ref[...])
            sup_ref[...] = jnp.dot(x_ref[...], hamilton,
                                   preferred_element_type=jnp.float32)
            stat_ref[...] = jnp.zeros_like(stat_ref)

        y = jnp.dot(adj_ref[...], sup_ref[...],
                    preferred_element_type=jnp.float32)
        out_ref[pl.ds(i * bm, bm), :] = y
        stat_ref[0:1, :] += jnp.sum(y, axis=0, keepdims=True)
        stat_ref[1:2, :] += jnp.sum(y * y, axis=0, keepdims=True)

        @pl.when(i == nblk - 1)
        def _epilogue():
            mean = stat_ref[0:1, :] / n
            var = stat_ref[1:2, :] / n - mean * mean
            scale = jax.lax.rsqrt(var + 1e-5) * g_ref[...]
            shift = b_ref[...] - mean * scale

            for j in range(nblk):
                yb = out_ref[j * bm:(j + 1) * bm, :]
                out_ref[j * bm:(j + 1) * bm, :] = jnp.tanh(yb * scale + shift)

    return fused


def kernel(input, adj, weight, gamma, beta):
    n, in_f = input.shape
    out_f = weight.shape[1]

    bm = 400
    nblk = n // bm
    return pl.pallas_call(
        _make_fused(n, out_f, bm),
        grid=(nblk,),
        in_specs=[
            pl.BlockSpec((n, in_f), lambda i: (0, 0)),
            pl.BlockSpec((in_f // 8, out_f), lambda i: (0, 0)),
            pl.BlockSpec((1, out_f), lambda i: (0, 0)),
            pl.BlockSpec((1, out_f), lambda i: (0, 0)),
            pl.BlockSpec((bm, n), lambda i: (i, 0)),
        ],
        out_specs=pl.BlockSpec((n, out_f), lambda i: (0, 0)),
        out_shape=jax.ShapeDtypeStruct((n, out_f), jnp.float32),
        scratch_shapes=[
            pltpu.VMEM((n, out_f), jnp.float32),
            pltpu.VMEM((8, out_f), jnp.float32),
        ],
    )(input, weight, gamma.reshape(1, out_f), beta.reshape(1, out_f), adj)
